# baseline (speedup 1.0000x reference)
import functools

import jax
import jax.numpy as jnp
from jax import lax
from jax.experimental import pallas as pl
from jax.experimental.pallas import tpu as pltpu
from jax.experimental.pallas import tpu_sc as plsc

N = 100000
E = 1600000
NH = 34
XD = 2
D = 40
QTR = 25000
ROWS_SH = 25088
TRASH = 25000
TILES = 16
T32 = 32
CHUNK = 128
SLAB = 512
EPS1 = 50176
NSLAB = EPS1 // SLAB
EP = EPS1 * T32
PADE = EP - E
FAR = 2 * N
BLKE = 2048
CAPB = 26
SROWS = 144
ZROWS = ROWS_SH // TILES

BLK = 5000


def _pre_body(x_ref, we1, be1, we2, be2, wm1, bm1, wm2, bm2, h_ref, msg_ref):
    x = x_ref[...]
    h = jnp.maximum(jnp.dot(x, we1[...], preferred_element_type=jnp.float32)
                    + be1[...], 0.0)
    h = jnp.dot(h, we2[...], preferred_element_type=jnp.float32) + be2[...]
    h_ref[...] = h
    m = jnp.maximum(jnp.dot(h, wm1[...], preferred_element_type=jnp.float32)
                    + bm1[...], 0.0)
    m = jnp.dot(m, wm2[...], preferred_element_type=jnp.float32) + bm2[...]
    msg_ref[...] = jnp.concatenate(
        [m, jnp.ones((BLK, 1), jnp.float32), jnp.zeros((BLK, D - NH - 1), jnp.float32)],
        axis=1)


def _full(shape):
    return pl.BlockSpec(shape, lambda i: (0, 0))


_pre_call = pl.pallas_call(
    _pre_body,
    grid=(N // BLK,),
    in_specs=[
        pl.BlockSpec((BLK, XD), lambda i: (i, 0)),
        _full((XD, NH)), _full((1, NH)), _full((NH, NH)), _full((1, NH)),
        _full((NH, NH)), _full((1, NH)), _full((NH, NH)), _full((1, NH)),
    ],
    out_specs=[
        pl.BlockSpec((BLK, NH), lambda i: (i, 0)),
        pl.BlockSpec((BLK, D), lambda i: (i, 0)),
    ],
    out_shape=[
        jax.ShapeDtypeStruct((N, NH), jnp.float32),
        jax.ShapeDtypeStruct((N, D), jnp.float32),
    ],
)


_mesh = plsc.VectorSubcoreMesh(core_axis_name="c", subcore_axis_name="s")


@functools.partial(
    pl.kernel,
    mesh=_mesh,
    out_type=[
        jax.ShapeDtypeStruct((T32, 4, CAPB, CHUNK, 16), jnp.int32),
        jax.ShapeDtypeStruct((T32, 4, CAPB, CHUNK, 16), jnp.int32),
        jax.ShapeDtypeStruct((T32, 4, 16), jnp.int32),
    ],
    scratch_types=[
        pltpu.VMEM((2, SLAB), jnp.int32),
        pltpu.VMEM((2, SLAB), jnp.int32),
        pltpu.VMEM((4, SROWS, 16), jnp.int32),
        pltpu.VMEM((4, SROWS, 16), jnp.int32),
        pltpu.VMEM((4, 16), jnp.int32),
        pltpu.SemaphoreType.DMA,
        pltpu.SemaphoreType.DMA,
    ],
    compiler_params=pltpu.CompilerParams(use_tc_tiling_on_sc=False,
                                         needs_layout_passes=False),
)
def _bucket_pass(ei_hbm, bsnd_hbm, bidx_hbm, cnts_hbm,
                 snd_v, rcv_v, stg_s, stg_i, cnts_v, sem0, sem1):
    c = lax.axis_index("c")
    s = lax.axis_index("s")
    t = s * 2 + c
    base = t * EPS1
    sems = (sem0, sem1)
    lane = lax.iota(jnp.int32, 16)
    zvec = jnp.zeros((16,), jnp.int32)
    tvec = jnp.full((16,), TRASH, jnp.int32)

    def fire(k, par):
        off = base + k * SLAB
        pltpu.async_copy(ei_hbm.at[pl.ds(off, SLAB)], snd_v.at[par], sems[par])
        pltpu.async_copy(ei_hbm.at[pl.ds(EP + off, SLAB)], rcv_v.at[par], sems[par])

    def wait_slab(par):
        pltpu.make_async_copy(ei_hbm.at[pl.ds(0, SLAB)], snd_v.at[par], sems[par]).wait()
        pltpu.make_async_copy(ei_hbm.at[pl.ds(0, SLAB)], rcv_v.at[par], sems[par]).wait()

    def flush_block(qq, hoff):
        blk = lax.shift_right_logical(hoff, 11)
        pltpu.sync_copy(stg_s.at[qq, pl.ds(0, CHUNK)], bsnd_hbm.at[t, qq, blk])
        pltpu.sync_copy(stg_i.at[qq, pl.ds(0, CHUNK)], bidx_hbm.at[t, qq, blk])

    def maybe_flush(qq, cnt, hoff):
        cond = cnt >= BLKE

        @pl.when(cond)
        def _():
            flush_block(qq, hoff)
            for g in range(8):
                vs = stg_s[qq, CHUNK + g]
                vi = stg_i[qq, CHUNK + g]
                stg_s[qq, g] = vs
                stg_i[qq, g] = vi

        step = BLKE * cond.astype(jnp.int32)
        return cnt - step, hoff + step

    def do_group(par, g, cnts):
        vs = snd_v[par, pl.ds(g * 16, 16)]
        vr = rcv_v[par, pl.ds(g * 16, 16)]
        q = ((vr >= QTR).astype(jnp.int32) + (vr >= 2 * QTR).astype(jnp.int32)
             + (vr >= 3 * QTR).astype(jnp.int32) + (vr >= N).astype(jnp.int32))
        local = vr - q * QTR
        out = []
        for qq in range(4):
            m = q == qq
            pos = plsc.cumsum(m.astype(jnp.int32)) - 1
            dest = cnts[qq] + pos
            row = lax.shift_right_arithmetic(dest, 4)
            col = jnp.bitwise_and(dest, 15)
            plsc.store_scatter(stg_s.at[qq], [row, col], vs, mask=m)
            plsc.store_scatter(stg_i.at[qq], [row, col], local, mask=m)
            inc = jnp.max(pos) + 1
            out.append(cnts[qq] + inc)
        return tuple(out)

    fire(0, 0)
    fire(1, 1)

    def body(k2, carry):
        cnts, hoffs = carry[:4], carry[4:]
        for par in range(2):
            k = 2 * k2 + par
            wait_slab(par)
            for g8 in range(4):
                for g in range(8):
                    cnts = do_group(par, g8 * 8 + g, cnts)
                new_c, new_h = [], []
                for qq in range(4):
                    cq, hq = maybe_flush(qq, cnts[qq], hoffs[qq])
                    new_c.append(cq)
                    new_h.append(hq)
                cnts, hoffs = tuple(new_c), tuple(new_h)

            @pl.when(k + 2 < NSLAB)
            def _():
                fire(k + 2, par)
        return cnts + hoffs

    carry = lax.fori_loop(0, NSLAB // 2, body,
                          (jnp.int32(0),) * 8)
    cnts, hoffs = carry[:4], carry[4:]

    for qq in range(4):
        cnt, hoff = cnts[qq], hoffs[qq]
        delta = jnp.bitwise_and(-cnt, 15)
        m = lane < delta
        dest = cnt + lane
        row = lax.shift_right_arithmetic(dest, 4)
        col = jnp.bitwise_and(dest, 15)
        plsc.store_scatter(stg_s.at[qq], [row, col], zvec, mask=m)
        plsc.store_scatter(stg_i.at[qq], [row, col], tvec, mask=m)
        cnt = cnt + delta
        total = hoff + cnt
        target = jnp.bitwise_and(total + (BLKE - 1), -BLKE)
        ngro = lax.shift_right_logical(target - total, 4)
        base_row = lax.shift_right_logical(cnt, 4)

        def padbody(tt, _):
            rw = zvec + (base_row + tt)
            plsc.store_scatter(stg_s.at[qq], [rw, lane], zvec, mask=lane < 16)
            plsc.store_scatter(stg_i.at[qq], [rw, lane], tvec, mask=lane < 16)
            return _

        lax.fori_loop(0, ngro, padbody, 0)
        cnt = cnt + lax.shift_left(ngro, 4)

        @pl.when(cnt >= BLKE)
        def _():
            flush_block(qq, hoff)

        hoff = hoff + BLKE * (cnt >= BLKE).astype(jnp.int32)
        cnts_v[qq] = zvec + hoff
    pltpu.sync_copy(cnts_v, cnts_hbm.at[t])


@functools.partial(
    pl.kernel,
    mesh=_mesh,
    out_type=jax.ShapeDtypeStruct((4, ROWS_SH, D), jnp.float32),
    scratch_types=[
        pltpu.VMEM((16, CHUNK), jnp.int32),
        pltpu.VMEM((16, CHUNK), jnp.int32),
        pltpu.VMEM((2, CHUNK, D), jnp.float32),
        pltpu.VMEM((4, 16), jnp.int32),
        pltpu.VMEM_SHARED((ROWS_SH, D), jnp.float32),
        [pltpu.SemaphoreType.DMA] * 2,
    ],
    compiler_params=pltpu.CompilerParams(use_tc_tiling_on_sc=False,
                                         needs_layout_passes=False),
)
def _edge_pass(bsnd_hbm, bidx_hbm, cnts_hbm, msg_hbm, zeros_hbm, out_hbm,
               snd_v, idx_v, rows_v, cnts_v, aggr_sh, sem_g):
    c = lax.axis_index("c")
    s = lax.axis_index("s")
    cvec = jnp.zeros((16,), jnp.int32) + c

    for pas in range(2):
        qt = 2 * c + pas
        pltpu.sync_copy(zeros_hbm, aggr_sh.at[pl.ds(s * ZROWS, ZROWS)])
        plsc.subcore_barrier()
        for so in range(2):
            src = 2 * s + so
            pltpu.sync_copy(cnts_hbm.at[src], cnts_v)
            ra = cnts_v[pas, pl.ds(0, 16)]
            rb = cnts_v[2 + pas, pl.ds(0, 16)]
            cntv = jnp.where(cvec == 0, ra, rb)
            nblk = lax.shift_right_logical(jnp.max(cntv), 11)

            def blk_body(bk, carry):
                pltpu.sync_copy(bsnd_hbm.at[src, qt, bk], snd_v)
                pltpu.sync_copy(bidx_hbm.at[src, qt, bk], idx_v)
                pltpu.async_copy(msg_hbm.at[snd_v.at[0]], rows_v.at[0],
                                 sem_g[0])
                for j in range(16):
                    b = j % 2
                    pltpu.make_async_copy(msg_hbm.at[snd_v.at[j]],
                                          rows_v.at[b], sem_g[b]).wait()
                    if j < 15:
                        pltpu.async_copy(msg_hbm.at[snd_v.at[j + 1]],
                                         rows_v.at[1 - b], sem_g[1 - b])
                    pltpu.sync_copy(rows_v.at[b], aggr_sh.at[idx_v.at[j]],
                                    add=True)
                return carry

            lax.fori_loop(0, nblk, blk_body, 0)
        plsc.subcore_barrier()
        pltpu.sync_copy(aggr_sh.at[pl.ds(s * ZROWS, ZROWS)],
                        out_hbm.at[qt, pl.ds(s * ZROWS, ZROWS)])
    plsc.subcore_barrier()


def _post_body(h_ref, a_ref, wn1, bn1, wn2, bn2, lns, lnb,
               wd1, bd1, wd2, bd2, wp1, bp1, wp2, bp2, wp3, bp3, out_ref):
    h = h_ref[...]
    a = a_ref[0]
    aggr = a[:, :NH]
    deg = a[:, NH:NH + 1]
    mean = aggr / jnp.maximum(deg, 1.0)
    u = jnp.concatenate([h, mean], axis=1)
    t = jnp.maximum(jnp.dot(u, wn1[...], preferred_element_type=jnp.float32)
                    + bn1[...], 0.0)
    t = jnp.dot(t, wn2[...], preferred_element_type=jnp.float32) + bn2[...]
    mu = jnp.mean(t, axis=1, keepdims=True)
    var = jnp.mean((t - mu) * (t - mu), axis=1, keepdims=True)
    t = (t - mu) * lax.rsqrt(var + 1e-5) * lns[...] + lnb[...]
    t = jnp.maximum(jnp.dot(t, wd1[...], preferred_element_type=jnp.float32)
                    + bd1[...], 0.0)
    t = jnp.dot(t, wd2[...], preferred_element_type=jnp.float32) + bd2[...]
    p = jnp.maximum(jnp.dot(t, wp1[...], preferred_element_type=jnp.float32)
                    + bp1[...], 0.0)
    p = jnp.maximum(jnp.dot(p, wp2[...], preferred_element_type=jnp.float32)
                    + bp2[...], 0.0)
    logits = jnp.dot(p, wp3[...], preferred_element_type=jnp.float32) + bp3[...]
    mx = jnp.max(logits, axis=1, keepdims=True)
    e = jnp.exp(logits - mx)
    out_ref[...] = e / jnp.sum(e, axis=1, keepdims=True)


_post_call = pl.pallas_call(
    _post_body,
    grid=(N // BLK,),
    in_specs=[
        pl.BlockSpec((BLK, NH), lambda i: (i, 0)),
        pl.BlockSpec((1, BLK, D), lambda i: (i // (QTR // BLK), i % (QTR // BLK), 0)),
        _full((2 * NH, NH)), _full((1, NH)), _full((NH, NH)), _full((1, NH)),
        _full((1, NH)), _full((1, NH)),
        _full((NH, NH)), _full((1, NH)), _full((NH, NH)), _full((1, NH)),
        _full((NH, NH)), _full((1, NH)), _full((NH, NH)), _full((1, NH)),
        _full((NH, 2)), _full((1, 2)),
    ],
    out_specs=pl.BlockSpec((BLK, 2), lambda i: (i, 0)),
    out_shape=jax.ShapeDtypeStruct((N, 2), jnp.float32),
)


def kernel(x, edge_index, W_enc1, b_enc1, W_enc2, b_enc2, W_msg1, b_msg1,
           W_msg2, b_msg2, W_nod1, b_nod1, W_nod2, b_nod2, ln_scale, ln_bias,
           W_dec1, b_dec1, W_dec2, b_dec2, W_p1, b_p1, W_p2, b_p2, W_p3, b_p3):
    r = lambda b: b.reshape(1, -1)
    h, msgpad = _pre_call(x, W_enc1, r(b_enc1), W_enc2, r(b_enc2),
                          W_msg1, r(b_msg1), W_msg2, r(b_msg2))
    ei_flat = jnp.concatenate([
        edge_index[0], jnp.zeros((PADE,), jnp.int32),
        edge_index[1], jnp.full((PADE,), FAR, jnp.int32)])
    bsnd, bidx, cnts = _bucket_pass(ei_flat)
    zeros = jnp.zeros((ZROWS, D), jnp.float32)
    aggr_raw = _edge_pass(bsnd.reshape(T32, 4, CAPB, 16, CHUNK),
                          bidx.reshape(T32, 4, CAPB, 16, CHUNK),
                          cnts, msgpad, zeros)
    return _post_call(h, aggr_raw, W_nod1, r(b_nod1), W_nod2, r(b_nod2),
                      r(ln_scale), r(ln_bias), W_dec1, r(b_dec1),
                      W_dec2, r(b_dec2), W_p1, r(b_p1), W_p2, r(b_p2),
                      W_p3, r(b_p3))


# speedup vs baseline: 1.0240x; 1.0240x over previous
import functools

import jax
import jax.numpy as jnp
from jax import lax
from jax.experimental import pallas as pl
from jax.experimental.pallas import tpu as pltpu
from jax.experimental.pallas import tpu_sc as plsc

N = 100000
E = 1600000
NH = 34
XD = 2
D = 40
QTR = 25000
ROWS_SH = 25088
TRASH = 25000
TILES = 16
T32 = 32
CHUNK = 128
SLAB = 512
EPS1 = 50176
NSLAB = EPS1 // SLAB
EP = EPS1 * T32
PADE = EP - E
FAR = 2 * N
BLKE = 2048
CAPB = 26
SROWS = 144
ZROWS = ROWS_SH // TILES

BLK = 5000


def _pre_body(x_ref, we1, be1, we2, be2, wm1, bm1, wm2, bm2, h_ref, msg_ref):
    x = x_ref[...]
    h = jnp.maximum(jnp.dot(x, we1[...], preferred_element_type=jnp.float32)
                    + be1[...], 0.0)
    h = jnp.dot(h, we2[...], preferred_element_type=jnp.float32) + be2[...]
    h_ref[...] = h
    m = jnp.maximum(jnp.dot(h, wm1[...], preferred_element_type=jnp.float32)
                    + bm1[...], 0.0)
    m = jnp.dot(m, wm2[...], preferred_element_type=jnp.float32) + bm2[...]
    msg_ref[...] = jnp.concatenate(
        [m, jnp.ones((BLK, 1), jnp.float32), jnp.zeros((BLK, D - NH - 1), jnp.float32)],
        axis=1)


def _full(shape):
    return pl.BlockSpec(shape, lambda i: (0, 0))


_pre_call = pl.pallas_call(
    _pre_body,
    grid=(N // BLK,),
    in_specs=[
        pl.BlockSpec((BLK, XD), lambda i: (i, 0)),
        _full((XD, NH)), _full((1, NH)), _full((NH, NH)), _full((1, NH)),
        _full((NH, NH)), _full((1, NH)), _full((NH, NH)), _full((1, NH)),
    ],
    out_specs=[
        pl.BlockSpec((BLK, NH), lambda i: (i, 0)),
        pl.BlockSpec((BLK, D), lambda i: (i, 0)),
    ],
    out_shape=[
        jax.ShapeDtypeStruct((N, NH), jnp.float32),
        jax.ShapeDtypeStruct((N, D), jnp.float32),
    ],
)


_mesh = plsc.VectorSubcoreMesh(core_axis_name="c", subcore_axis_name="s")


@functools.partial(
    pl.kernel,
    mesh=_mesh,
    out_type=[
        jax.ShapeDtypeStruct((T32, 4, CAPB, CHUNK, 16), jnp.int32),
        jax.ShapeDtypeStruct((T32, 4, CAPB, CHUNK, 16), jnp.int32),
        jax.ShapeDtypeStruct((T32, 4, 16), jnp.int32),
    ],
    scratch_types=[
        pltpu.VMEM((2, SLAB), jnp.int32),
        pltpu.VMEM((2, SLAB), jnp.int32),
        pltpu.VMEM((4, SROWS, 16), jnp.int32),
        pltpu.VMEM((4, SROWS, 16), jnp.int32),
        pltpu.VMEM((4, 16), jnp.int32),
        pltpu.SemaphoreType.DMA,
        pltpu.SemaphoreType.DMA,
    ],
    compiler_params=pltpu.CompilerParams(use_tc_tiling_on_sc=False,
                                         needs_layout_passes=False),
)
def _bucket_pass(ei_hbm, bsnd_hbm, bidx_hbm, cnts_hbm,
                 snd_v, rcv_v, stg_s, stg_i, cnts_v, sem0, sem1):
    c = lax.axis_index("c")
    s = lax.axis_index("s")
    t = s * 2 + c
    base = t * EPS1
    sems = (sem0, sem1)
    lane = lax.iota(jnp.int32, 16)
    zvec = jnp.zeros((16,), jnp.int32)
    tvec = jnp.full((16,), TRASH, jnp.int32)

    def fire(k, par):
        off = base + k * SLAB
        pltpu.async_copy(ei_hbm.at[pl.ds(off, SLAB)], snd_v.at[par], sems[par])
        pltpu.async_copy(ei_hbm.at[pl.ds(EP + off, SLAB)], rcv_v.at[par], sems[par])

    def wait_slab(par):
        pltpu.make_async_copy(ei_hbm.at[pl.ds(0, SLAB)], snd_v.at[par], sems[par]).wait()
        pltpu.make_async_copy(ei_hbm.at[pl.ds(0, SLAB)], rcv_v.at[par], sems[par]).wait()

    def flush_block(qq, hoff):
        blk = lax.shift_right_logical(hoff, 11)
        pltpu.sync_copy(stg_s.at[qq, pl.ds(0, CHUNK)], bsnd_hbm.at[t, qq, blk])
        pltpu.sync_copy(stg_i.at[qq, pl.ds(0, CHUNK)], bidx_hbm.at[t, qq, blk])

    def maybe_flush(qq, cnt, hoff):
        cond = cnt >= BLKE

        @pl.when(cond)
        def _():
            flush_block(qq, hoff)
            for g in range(8):
                vs = stg_s[qq, CHUNK + g]
                vi = stg_i[qq, CHUNK + g]
                stg_s[qq, g] = vs
                stg_i[qq, g] = vi

        step = BLKE * cond.astype(jnp.int32)
        return cnt - step, hoff + step

    def do_group(par, g, cnts):
        vs = snd_v[par, pl.ds(g * 16, 16)]
        vr = rcv_v[par, pl.ds(g * 16, 16)]
        q = ((vr >= QTR).astype(jnp.int32) + (vr >= 2 * QTR).astype(jnp.int32)
             + (vr >= 3 * QTR).astype(jnp.int32) + (vr >= N).astype(jnp.int32))
        local = vr - q * QTR
        out = []
        for qq in range(4):
            m = q == qq
            pos = plsc.cumsum(m.astype(jnp.int32)) - 1
            dest = cnts[qq] + pos
            row = lax.shift_right_arithmetic(dest, 4)
            col = jnp.bitwise_and(dest, 15)
            plsc.store_scatter(stg_s.at[qq], [row, col], vs, mask=m)
            plsc.store_scatter(stg_i.at[qq], [row, col], local, mask=m)
            inc = jnp.max(pos) + 1
            out.append(cnts[qq] + inc)
        return tuple(out)

    fire(0, 0)
    fire(1, 1)

    def body(k2, carry):
        cnts, hoffs = carry[:4], carry[4:]
        for par in range(2):
            k = 2 * k2 + par
            wait_slab(par)
            for g8 in range(4):
                for g in range(8):
                    cnts = do_group(par, g8 * 8 + g, cnts)
                new_c, new_h = [], []
                for qq in range(4):
                    cq, hq = maybe_flush(qq, cnts[qq], hoffs[qq])
                    new_c.append(cq)
                    new_h.append(hq)
                cnts, hoffs = tuple(new_c), tuple(new_h)

            @pl.when(k + 2 < NSLAB)
            def _():
                fire(k + 2, par)
        return cnts + hoffs

    carry = lax.fori_loop(0, NSLAB // 2, body,
                          (jnp.int32(0),) * 8)
    cnts, hoffs = carry[:4], carry[4:]

    for qq in range(4):
        cnt, hoff = cnts[qq], hoffs[qq]
        delta = jnp.bitwise_and(-cnt, 15)
        m = lane < delta
        dest = cnt + lane
        row = lax.shift_right_arithmetic(dest, 4)
        col = jnp.bitwise_and(dest, 15)
        plsc.store_scatter(stg_s.at[qq], [row, col], zvec, mask=m)
        plsc.store_scatter(stg_i.at[qq], [row, col], tvec, mask=m)
        cnt = cnt + delta
        total = hoff + cnt
        target = jnp.bitwise_and(total + (BLKE - 1), -BLKE)
        ngro = lax.shift_right_logical(target - total, 4)
        base_row = lax.shift_right_logical(cnt, 4)

        def padbody(tt, _):
            rw = zvec + (base_row + tt)
            plsc.store_scatter(stg_s.at[qq], [rw, lane], zvec, mask=lane < 16)
            plsc.store_scatter(stg_i.at[qq], [rw, lane], tvec, mask=lane < 16)
            return _

        lax.fori_loop(0, ngro, padbody, 0)
        cnt = cnt + lax.shift_left(ngro, 4)

        @pl.when(cnt >= BLKE)
        def _():
            flush_block(qq, hoff)

        hoff = hoff + BLKE * (cnt >= BLKE).astype(jnp.int32)
        cnts_v[qq] = zvec + hoff
    pltpu.sync_copy(cnts_v, cnts_hbm.at[t])


@functools.partial(
    pl.kernel,
    mesh=_mesh,
    out_type=jax.ShapeDtypeStruct((4, ROWS_SH, D), jnp.float32),
    scratch_types=[
        pltpu.VMEM((16, CHUNK), jnp.int32),
        pltpu.VMEM((16, CHUNK), jnp.int32),
        pltpu.VMEM((2, CHUNK, D), jnp.float32),
        pltpu.VMEM((CHUNK,), jnp.int32),
        pltpu.VMEM((CHUNK,), jnp.int32),
        pltpu.VMEM((CHUNK,), jnp.int32),
        pltpu.VMEM((CHUNK,), jnp.int32),
        pltpu.VMEM((4, 16), jnp.int32),
        pltpu.VMEM_SHARED((ROWS_SH, D), jnp.float32),
        [pltpu.SemaphoreType.DMA] * 2,
    ],
    compiler_params=pltpu.CompilerParams(use_tc_tiling_on_sc=False,
                                         needs_layout_passes=False),
)
def _edge_pass(bsnd_hbm, bidx_hbm, cnts_hbm, msg_hbm, zeros_hbm, out_hbm,
               snd_v, idx_v, rows_v, sf0, sf1, if0, if1, cnts_v, aggr_sh,
               sem_g):
    c = lax.axis_index("c")
    s = lax.axis_index("s")
    cvec = jnp.zeros((16,), jnp.int32) + c

    for pas in range(2):
        qt = 2 * c + pas
        pltpu.sync_copy(zeros_hbm, aggr_sh.at[pl.ds(s * ZROWS, ZROWS)])
        plsc.subcore_barrier()
        for so in range(2):
            src = 2 * s + so
            pltpu.sync_copy(cnts_hbm.at[src], cnts_v)
            ra = cnts_v[pas, pl.ds(0, 16)]
            rb = cnts_v[2 + pas, pl.ds(0, 16)]
            cntv = jnp.where(cvec == 0, ra, rb)
            nblk = lax.shift_right_logical(jnp.max(cntv), 11)

            sf = (sf0, sf1)
            fi = (if0, if1)

            def stage_idx(j, b):
                for l in range(CHUNK // 16):
                    sf[b][pl.ds(l * 16, 16)] = snd_v[j, pl.ds(l * 16, 16)]
                    fi[b][pl.ds(l * 16, 16)] = idx_v[j, pl.ds(l * 16, 16)]

            def blk_body(bk, carry):
                pltpu.sync_copy(bsnd_hbm.at[src, qt, bk], snd_v)
                pltpu.sync_copy(bidx_hbm.at[src, qt, bk], idx_v)
                stage_idx(0, 0)
                pltpu.async_copy(msg_hbm.at[sf0], rows_v.at[0],
                                 sem_g[0])
                for j in range(16):
                    b = j % 2
                    if j < 15:
                        stage_idx(j + 1, 1 - b)
                        pltpu.async_copy(msg_hbm.at[sf[1 - b]],
                                         rows_v.at[1 - b], sem_g[1 - b])
                    pltpu.make_async_copy(msg_hbm.at[sf[b]],
                                          rows_v.at[b], sem_g[b]).wait()
                    pltpu.sync_copy(rows_v.at[b], aggr_sh.at[fi[b]],
                                    add=True)
                return carry

            lax.fori_loop(0, nblk, blk_body, 0)
        plsc.subcore_barrier()
        pltpu.sync_copy(aggr_sh.at[pl.ds(s * ZROWS, ZROWS)],
                        out_hbm.at[qt, pl.ds(s * ZROWS, ZROWS)])
    plsc.subcore_barrier()


def _post_body(h_ref, a_ref, wn1, bn1, wn2, bn2, lns, lnb,
               wd1, bd1, wd2, bd2, wp1, bp1, wp2, bp2, wp3, bp3, out_ref):
    h = h_ref[...]
    a = a_ref[0]
    aggr = a[:, :NH]
    deg = a[:, NH:NH + 1]
    mean = aggr / jnp.maximum(deg, 1.0)
    u = jnp.concatenate([h, mean], axis=1)
    t = jnp.maximum(jnp.dot(u, wn1[...], preferred_element_type=jnp.float32)
                    + bn1[...], 0.0)
    t = jnp.dot(t, wn2[...], preferred_element_type=jnp.float32) + bn2[...]
    mu = jnp.mean(t, axis=1, keepdims=True)
    var = jnp.mean((t - mu) * (t - mu), axis=1, keepdims=True)
    t = (t - mu) * lax.rsqrt(var + 1e-5) * lns[...] + lnb[...]
    t = jnp.maximum(jnp.dot(t, wd1[...], preferred_element_type=jnp.float32)
                    + bd1[...], 0.0)
    t = jnp.dot(t, wd2[...], preferred_element_type=jnp.float32) + bd2[...]
    p = jnp.maximum(jnp.dot(t, wp1[...], preferred_element_type=jnp.float32)
                    + bp1[...], 0.0)
    p = jnp.maximum(jnp.dot(p, wp2[...], preferred_element_type=jnp.float32)
                    + bp2[...], 0.0)
    logits = jnp.dot(p, wp3[...], preferred_element_type=jnp.float32) + bp3[...]
    mx = jnp.max(logits, axis=1, keepdims=True)
    e = jnp.exp(logits - mx)
    out_ref[...] = e / jnp.sum(e, axis=1, keepdims=True)


_post_call = pl.pallas_call(
    _post_body,
    grid=(N // BLK,),
    in_specs=[
        pl.BlockSpec((BLK, NH), lambda i: (i, 0)),
        pl.BlockSpec((1, BLK, D), lambda i: (i // (QTR // BLK), i % (QTR // BLK), 0)),
        _full((2 * NH, NH)), _full((1, NH)), _full((NH, NH)), _full((1, NH)),
        _full((1, NH)), _full((1, NH)),
        _full((NH, NH)), _full((1, NH)), _full((NH, NH)), _full((1, NH)),
        _full((NH, NH)), _full((1, NH)), _full((NH, NH)), _full((1, NH)),
        _full((NH, 2)), _full((1, 2)),
    ],
    out_specs=pl.BlockSpec((BLK, 2), lambda i: (i, 0)),
    out_shape=jax.ShapeDtypeStruct((N, 2), jnp.float32),
)


def kernel(x, edge_index, W_enc1, b_enc1, W_enc2, b_enc2, W_msg1, b_msg1,
           W_msg2, b_msg2, W_nod1, b_nod1, W_nod2, b_nod2, ln_scale, ln_bias,
           W_dec1, b_dec1, W_dec2, b_dec2, W_p1, b_p1, W_p2, b_p2, W_p3, b_p3):
    r = lambda b: b.reshape(1, -1)
    h, msgpad = _pre_call(x, W_enc1, r(b_enc1), W_enc2, r(b_enc2),
                          W_msg1, r(b_msg1), W_msg2, r(b_msg2))
    ei_flat = jnp.concatenate([
        edge_index[0], jnp.zeros((PADE,), jnp.int32),
        edge_index[1], jnp.full((PADE,), FAR, jnp.int32)])
    bsnd, bidx, cnts = _bucket_pass(ei_flat)
    zeros = jnp.zeros((ZROWS, D), jnp.float32)
    aggr_raw = _edge_pass(bsnd.reshape(T32, 4, CAPB, 16, CHUNK),
                          bidx.reshape(T32, 4, CAPB, 16, CHUNK),
                          cnts, msgpad, zeros)
    return _post_call(h, aggr_raw, W_nod1, r(b_nod1), W_nod2, r(b_nod2),
                      r(ln_scale), r(ln_bias), W_dec1, r(b_dec1),
                      W_dec2, r(b_dec2), W_p1, r(b_p1), W_p2, r(b_p2),
                      W_p3, r(b_p3))


# speedup vs baseline: 2.2886x; 2.2349x over previous
import functools

import jax
import jax.numpy as jnp
from jax import lax
from jax.experimental import pallas as pl
from jax.experimental.pallas import tpu as pltpu
from jax.experimental.pallas import tpu_sc as plsc

N = 100000
E = 1600000
NH = 34
XD = 2
D = 40
HALF = 50000
ROWS_SH = 50048
TRASH = 50000
TILES = 16
EPT = E // TILES
CHUNK = 128
NFULL = EPT // CHUNK
REM = EPT - NFULL * CHUNK
ZROWS = ROWS_SH // TILES

BLK = 5000


def _pre_body(x_ref, we1, be1, we2, be2, wm1, bm1, wm2, bm2, h_ref, msg_ref):
    x = x_ref[...]
    h = jnp.maximum(jnp.dot(x, we1[...], preferred_element_type=jnp.float32)
                    + be1[...], 0.0)
    h = jnp.dot(h, we2[...], preferred_element_type=jnp.float32) + be2[...]
    h_ref[...] = h
    m = jnp.maximum(jnp.dot(h, wm1[...], preferred_element_type=jnp.float32)
                    + bm1[...], 0.0)
    m = jnp.dot(m, wm2[...], preferred_element_type=jnp.float32) + bm2[...]
    msg_ref[...] = jnp.concatenate(
        [m, jnp.ones((BLK, 1), jnp.float32), jnp.zeros((BLK, D - NH - 1), jnp.float32)],
        axis=1)


def _full(shape):
    return pl.BlockSpec(shape, lambda i: (0, 0))


_pre_call = pl.pallas_call(
    _pre_body,
    grid=(N // BLK,),
    in_specs=[
        pl.BlockSpec((BLK, XD), lambda i: (i, 0)),
        _full((XD, NH)), _full((1, NH)), _full((NH, NH)), _full((1, NH)),
        _full((NH, NH)), _full((1, NH)), _full((NH, NH)), _full((1, NH)),
    ],
    out_specs=[
        pl.BlockSpec((BLK, NH), lambda i: (i, 0)),
        pl.BlockSpec((BLK, D), lambda i: (i, 0)),
    ],
    out_shape=[
        jax.ShapeDtypeStruct((N, NH), jnp.float32),
        jax.ShapeDtypeStruct((N, D), jnp.float32),
    ],
)


_mesh = plsc.VectorSubcoreMesh(core_axis_name="c", subcore_axis_name="s")


@functools.partial(
    pl.kernel,
    mesh=_mesh,
    out_type=jax.ShapeDtypeStruct((2, ROWS_SH, D), jnp.float32),
    scratch_types=[
        pltpu.VMEM((CHUNK,), jnp.int32),
        pltpu.VMEM((CHUNK,), jnp.int32),
        pltpu.VMEM((CHUNK,), jnp.int32),
        pltpu.VMEM((CHUNK, D), jnp.float32),
        pltpu.VMEM_SHARED((ROWS_SH, D), jnp.float32),
        pltpu.SemaphoreType.DMA,
    ],
    compiler_params=pltpu.CompilerParams(use_tc_tiling_on_sc=False),
)
def _edge_pass(ei_hbm, msg_hbm, zeros_hbm, out_hbm,
               snd_v, rcv_v, ridx_v, rows_v, aggr_sh, sem):
    c = lax.axis_index("c")
    s = lax.axis_index("s")
    lo = c * HALF

    pltpu.sync_copy(zeros_hbm, aggr_sh.at[pl.ds(s * ZROWS, ZROWS)])
    plsc.subcore_barrier()

    base = s * EPT

    def do_chunk(off, first_valid):
        pltpu.sync_copy(ei_hbm.at[pl.ds(off, CHUNK)], snd_v)
        pltpu.sync_copy(ei_hbm.at[pl.ds(E + off, CHUNK)], rcv_v)
        pltpu.async_copy(msg_hbm.at[snd_v], rows_v, sem).wait()
        for j in range(CHUNK // 16):
            r = rcv_v[pl.ds(j * 16, 16)]
            ok = (r >= lo) & (r < lo + HALF)
            if first_valid is not None:
                lane = lax.iota(jnp.int32, 16) + (j * 16)
                ok = ok & (lane >= first_valid)
            ridx_v[pl.ds(j * 16, 16)] = jnp.where(ok, r - lo, TRASH)
        pltpu.sync_copy(rows_v, aggr_sh.at[ridx_v], add=True)

    def body(k, carry):
        do_chunk(base + k * CHUNK, None)
        return carry

    lax.fori_loop(0, NFULL, body, 0)
    do_chunk(base + EPT - CHUNK, CHUNK - REM)

    plsc.subcore_barrier()
    pltpu.sync_copy(aggr_sh.at[pl.ds(s * ZROWS, ZROWS)],
                    out_hbm.at[c, pl.ds(s * ZROWS, ZROWS)])


def _post_body(h_ref, a_ref, wn1, bn1, wn2, bn2, lns, lnb,
               wd1, bd1, wd2, bd2, wp1, bp1, wp2, bp2, wp3, bp3, out_ref):
    h = h_ref[...]
    a = a_ref[0]
    aggr = a[:, :NH]
    deg = a[:, NH:NH + 1]
    mean = aggr / jnp.maximum(deg, 1.0)
    u = jnp.concatenate([h, mean], axis=1)
    t = jnp.maximum(jnp.dot(u, wn1[...], preferred_element_type=jnp.float32)
                    + bn1[...], 0.0)
    t = jnp.dot(t, wn2[...], preferred_element_type=jnp.float32) + bn2[...]
    mu = jnp.mean(t, axis=1, keepdims=True)
    var = jnp.mean((t - mu) * (t - mu), axis=1, keepdims=True)
    t = (t - mu) * lax.rsqrt(var + 1e-5) * lns[...] + lnb[...]
    t = jnp.maximum(jnp.dot(t, wd1[...], preferred_element_type=jnp.float32)
                    + bd1[...], 0.0)
    t = jnp.dot(t, wd2[...], preferred_element_type=jnp.float32) + bd2[...]
    p = jnp.maximum(jnp.dot(t, wp1[...], preferred_element_type=jnp.float32)
                    + bp1[...], 0.0)
    p = jnp.maximum(jnp.dot(p, wp2[...], preferred_element_type=jnp.float32)
                    + bp2[...], 0.0)
    logits = jnp.dot(p, wp3[...], preferred_element_type=jnp.float32) + bp3[...]
    mx = jnp.max(logits, axis=1, keepdims=True)
    e = jnp.exp(logits - mx)
    out_ref[...] = e / jnp.sum(e, axis=1, keepdims=True)


_post_call = pl.pallas_call(
    _post_body,
    grid=(N // BLK,),
    in_specs=[
        pl.BlockSpec((BLK, NH), lambda i: (i, 0)),
        pl.BlockSpec((1, BLK, D), lambda i: (i // (HALF // BLK), i % (HALF // BLK), 0)),
        _full((2 * NH, NH)), _full((1, NH)), _full((NH, NH)), _full((1, NH)),
        _full((1, NH)), _full((1, NH)),
        _full((NH, NH)), _full((1, NH)), _full((NH, NH)), _full((1, NH)),
        _full((NH, NH)), _full((1, NH)), _full((NH, NH)), _full((1, NH)),
        _full((NH, 2)), _full((1, 2)),
    ],
    out_specs=pl.BlockSpec((BLK, 2), lambda i: (i, 0)),
    out_shape=jax.ShapeDtypeStruct((N, 2), jnp.float32),
)


def kernel(x, edge_index, W_enc1, b_enc1, W_enc2, b_enc2, W_msg1, b_msg1,
           W_msg2, b_msg2, W_nod1, b_nod1, W_nod2, b_nod2, ln_scale, ln_bias,
           W_dec1, b_dec1, W_dec2, b_dec2, W_p1, b_p1, W_p2, b_p2, W_p3, b_p3):
    r = lambda b: b.reshape(1, -1)
    h, msgpad = _pre_call(x, W_enc1, r(b_enc1), W_enc2, r(b_enc2),
                          W_msg1, r(b_msg1), W_msg2, r(b_msg2))
    zeros = jnp.zeros((ZROWS, D), jnp.float32)
    aggr_raw = _edge_pass(edge_index.reshape(-1), msgpad, zeros)
    return _post_call(h, aggr_raw, W_nod1, r(b_nod1), W_nod2, r(b_nod2),
                      r(ln_scale), r(ln_bias), W_dec1, r(b_dec1),
                      W_dec2, r(b_dec2), W_p1, r(b_p1), W_p2, r(b_p2),
                      W_p3, r(b_p3))
